# Initial kernel scaffold; baseline (speedup 1.0000x reference)
import jax
import jax.numpy as jnp
from jax.experimental import pallas as pl


def kernel(features, edge_index, W, b):
    raise NotImplementedError("write your pallas kernel here")


# speedup vs baseline: 3.7164x; 3.7164x over previous
import dataclasses
import functools

import jax
import jax.numpy as jnp
from jax import lax
from jax.experimental import pallas as pl
from jax.experimental.pallas import tpu as pltpu
from jax.experimental.pallas import tpu_sc as plsc

_N = 10000
_D = 128
_NPAD = 10240
_NC, _NS, _L = 2, 16, 16
_NW = _NC * _NS
_CH = 80
_EPAD = _NW * _CH * 128
_STRIPE = _NPAD // _NS

_mesh = plsc.VectorSubcoreMesh(core_axis_name="c", subcore_axis_name="s")

_cp = pltpu.CompilerParams()
if "needs_layout_passes" in pltpu.CompilerParams.__dataclass_fields__:
    _cp = dataclasses.replace(_cp, needs_layout_passes=False)


def _deg_partials(dst2d):
    @functools.partial(
        pl.kernel,
        out_type=jax.ShapeDtypeStruct((_NW, _NPAD), jnp.float32),
        mesh=_mesh,
        compiler_params=_cp,
        scratch_types=[
            pltpu.VMEM((_CH, 128), jnp.int32),
            pltpu.VMEM((_NPAD,), jnp.float32),
        ],
    )
    def k(dst_hbm, out_hbm, idx_v, hist_v):
        c = lax.axis_index("c")
        s = lax.axis_index("s")
        wid = c * _NS + s
        pltpu.sync_copy(dst_hbm.at[pl.ds(wid * _CH, _CH)], idx_v)

        @pl.loop(0, _NPAD, step=_L)
        def _zero(i):
            hist_v[pl.ds(i, _L)] = jnp.zeros((_L,), jnp.float32)

        ones = jnp.ones((_L,), jnp.float32)

        @pl.loop(0, _CH)
        def _row(r):
            @pl.loop(0, 128, step=_L)
            def _vec(t):
                idx = idx_v[r, pl.ds(t, _L)]
                plsc.addupdate_scatter(hist_v, [idx], ones)

        pltpu.sync_copy(hist_v, out_hbm.at[wid])

    return k(dst2d)


def _hop(h, src2d, dst2d):
    @functools.partial(
        pl.kernel,
        out_type=jax.ShapeDtypeStruct((_NC, _NPAD, _D), jnp.float32),
        mesh=_mesh,
        scratch_types=[
            pltpu.VMEM((_CH // 2, 128), jnp.int32),
            pltpu.VMEM((_CH // 2, 128), jnp.int32),
            pltpu.VMEM((128, _D), jnp.float32),
            pltpu.VMEM((128, _D), jnp.float32),
            pltpu.VMEM_SHARED((_NPAD, _D), jnp.float32),
            pltpu.SemaphoreType.DMA,
            pltpu.SemaphoreType.DMA,
        ],
    )
    def k(h_hbm, src_hbm, dst_hbm, out_hbm, si_v, di_v, rows0, rows1, acc, sem0, sem1):
        c = lax.axis_index("c")
        s = lax.axis_index("s")
        wid = c * _NS + s
        hch = _CH // 2

        @pl.loop(0, 128)
        def _zrow(r):
            @pl.loop(0, _D, step=_L)
            def _zvec(t):
                rows0[r, pl.ds(t, _L)] = jnp.zeros((_L,), jnp.float32)

        @pl.loop(0, _STRIPE, step=128)
        def _zacc(i):
            pltpu.sync_copy(rows0, acc.at[pl.ds(s * _STRIPE + i, 128)])

        plsc.subcore_barrier()

        @pl.loop(0, 2)
        def _half(p):
            base = wid * _CH + p * hch
            pltpu.sync_copy(src_hbm.at[pl.ds(base, hch)], si_v)
            pltpu.sync_copy(dst_hbm.at[pl.ds(base, hch)], di_v)
            pltpu.async_copy(h_hbm.at[si_v.at[0]], rows0, sem0)

            @pl.loop(0, hch, step=2)
            def _edge(j):
                pltpu.async_copy(h_hbm.at[si_v.at[j + 1]], rows1, sem1)
                pltpu.make_async_copy(h_hbm.at[si_v.at[0]], rows0, sem0).wait()
                pltpu.sync_copy(rows0, acc.at[di_v.at[j]], add=True)

                @pl.when(j + 2 < hch)
                def _pref():
                    pltpu.async_copy(h_hbm.at[si_v.at[j + 2]], rows0, sem0)

                pltpu.make_async_copy(h_hbm.at[si_v.at[0]], rows1, sem1).wait()
                pltpu.sync_copy(rows1, acc.at[di_v.at[j + 1]], add=True)

        plsc.subcore_barrier()
        pltpu.sync_copy(
            acc.at[pl.ds(s * _STRIPE, _STRIPE)],
            out_hbm.at[c].at[pl.ds(s * _STRIPE, _STRIPE)],
        )

    return k(h, src2d, dst2d)


def _matmul(x, w):
    def body(x_ref, w_ref, o_ref):
        o_ref[...] = jnp.dot(x_ref[...], w_ref[...], preferred_element_type=jnp.float32)

    return pl.pallas_call(
        body, out_shape=jax.ShapeDtypeStruct((_NPAD, _D), jnp.float32)
    )(x, w)


def _prep(parts, y):
    def body(p_ref, y_ref, h1_ref, n_ref):
        deg = jnp.sum(p_ref[...], axis=0)
        norm = lax.rsqrt(jnp.maximum(deg, 1.0))
        n_col = norm[:, None]
        n_ref[...] = n_col
        h1_ref[...] = y_ref[...] * n_col

    return pl.pallas_call(
        body,
        out_shape=[
            jax.ShapeDtypeStruct((_NPAD, _D), jnp.float32),
            jax.ShapeDtypeStruct((_NPAD, 1), jnp.float32),
        ],
    )(parts, y)


def _mid(zp, n_col):
    def body(z_ref, n_ref, o_ref):
        n2 = n_ref[...] * n_ref[...]
        o_ref[...] = (z_ref[0] + z_ref[1]) * n2

    return pl.pallas_call(
        body, out_shape=jax.ShapeDtypeStruct((_NPAD, _D), jnp.float32)
    )(zp, n_col)


def _final(up, n_col, b2d):
    def body(u_ref, n_ref, b_ref, o_ref):
        u = (u_ref[0] + u_ref[1]) * n_ref[...]
        o_ref[...] = u[:_N, :] + b_ref[...]

    return pl.pallas_call(
        body, out_shape=jax.ShapeDtypeStruct((_N, _D), jnp.float32)
    )(up, n_col, b2d)


def kernel(features, edge_index, W, b):
    src = edge_index[0]
    dst = edge_index[1]
    e = src.shape[0]
    fill = jnp.full((_EPAD - e,), _N, jnp.int32)
    src2d = jnp.concatenate([src, fill]).reshape(_NW * _CH, 128)
    dst2d = jnp.concatenate([dst, fill]).reshape(_NW * _CH, 128)
    x_pad = jnp.pad(features, ((0, _NPAD - _N), (0, 0)))

    parts = _deg_partials(dst2d)
    y = _matmul(x_pad, W)
    h1, n_col = _prep(parts, y)
    zp = _hop(h1, src2d, dst2d)
    h2 = _mid(zp, n_col)
    up = _hop(h2, src2d, dst2d)
    return _final(up, n_col, jnp.reshape(b, (1, _D)))


# speedup vs baseline: 11.0969x; 2.9859x over previous
import dataclasses
import functools

import jax
import jax.numpy as jnp
from jax import lax
from jax.experimental import pallas as pl
from jax.experimental.pallas import tpu as pltpu
from jax.experimental.pallas import tpu_sc as plsc

_N = 10000
_D = 128
_NPAD = 10240
_NC, _NS, _L = 2, 16, 16
_NW = _NC * _NS
_CH = 80
_EPAD = _NW * _CH * 128
_STRIPE = _NPAD // _NS

_mesh = plsc.VectorSubcoreMesh(core_axis_name="c", subcore_axis_name="s")

_cp = pltpu.CompilerParams()
if "needs_layout_passes" in pltpu.CompilerParams.__dataclass_fields__:
    _cp = dataclasses.replace(_cp, needs_layout_passes=False)


def _deg_partials(dst2d):
    @functools.partial(
        pl.kernel,
        out_type=jax.ShapeDtypeStruct((_NW, _NPAD), jnp.float32),
        mesh=_mesh,
        compiler_params=_cp,
        scratch_types=[
            pltpu.VMEM((_CH, 128), jnp.int32),
            pltpu.VMEM((_NPAD,), jnp.float32),
        ],
    )
    def k(dst_hbm, out_hbm, idx_v, hist_v):
        c = lax.axis_index("c")
        s = lax.axis_index("s")
        wid = c * _NS + s
        pltpu.sync_copy(dst_hbm.at[pl.ds(wid * _CH, _CH)], idx_v)

        @pl.loop(0, _NPAD, step=_L)
        def _zero(i):
            hist_v[pl.ds(i, _L)] = jnp.zeros((_L,), jnp.float32)

        ones = jnp.ones((_L,), jnp.float32)

        @pl.loop(0, _CH)
        def _row(r):
            @pl.loop(0, 128, step=_L)
            def _vec(t):
                idx = idx_v[r, pl.ds(t, _L)]
                plsc.addupdate_scatter(hist_v, [idx], ones)

        pltpu.sync_copy(hist_v, out_hbm.at[wid])

    return k(dst2d)


def _hop(h, src2d, dst2d):
    @functools.partial(
        pl.kernel,
        out_type=jax.ShapeDtypeStruct((_NC, _NPAD, _D), jnp.float32),
        mesh=_mesh,
        scratch_types=[
            pltpu.VMEM((_CH // 2, 128), jnp.int32),
            pltpu.VMEM((_CH // 2, 128), jnp.int32),
            pltpu.VMEM((128, _D), jnp.float32),
            pltpu.VMEM((128, _D), jnp.float32),
            pltpu.VMEM_SHARED((_NPAD, _D), jnp.float32),
            pltpu.SemaphoreType.DMA,
            pltpu.SemaphoreType.DMA,
        ],
    )
    def k(h_hbm, src_hbm, dst_hbm, out_hbm, si_v, di_v, rows0, rows1, acc, sem0, sem1):
        c = lax.axis_index("c")
        s = lax.axis_index("s")
        wid = c * _NS + s
        hch = _CH // 2

        @pl.loop(0, 128)
        def _zrow(r):
            @pl.loop(0, _D, step=_L)
            def _zvec(t):
                rows0[r, pl.ds(t, _L)] = jnp.zeros((_L,), jnp.float32)

        @pl.loop(0, _STRIPE, step=128)
        def _zacc(i):
            pltpu.sync_copy(rows0, acc.at[pl.ds(s * _STRIPE + i, 128)])

        plsc.subcore_barrier()

        @pl.loop(0, 2)
        def _half(p):
            base = wid * _CH + p * hch
            pltpu.sync_copy(src_hbm.at[pl.ds(base, hch)], si_v)
            pltpu.sync_copy(dst_hbm.at[pl.ds(base, hch)], di_v)
            pltpu.async_copy(h_hbm.at[si_v.at[0]], rows0, sem0)

            @pl.loop(0, hch, step=2)
            def _edge(j):
                pltpu.async_copy(h_hbm.at[si_v.at[j + 1]], rows1, sem1)
                pltpu.make_async_copy(h_hbm.at[si_v.at[0]], rows0, sem0).wait()
                pltpu.sync_copy(rows0, acc.at[di_v.at[j]], add=True)

                @pl.when(j + 2 < hch)
                def _pref():
                    pltpu.async_copy(h_hbm.at[si_v.at[j + 2]], rows0, sem0)

                pltpu.make_async_copy(h_hbm.at[si_v.at[0]], rows1, sem1).wait()
                pltpu.sync_copy(rows1, acc.at[di_v.at[j + 1]], add=True)

        plsc.subcore_barrier()
        pltpu.sync_copy(
            acc.at[pl.ds(s * _STRIPE, _STRIPE)],
            out_hbm.at[c].at[pl.ds(s * _STRIPE, _STRIPE)],
        )

    return k(h, src2d, dst2d)


def _matmul(x, w):
    def body(x_ref, w_ref, o_ref):
        o_ref[...] = jnp.dot(x_ref[...], w_ref[...], preferred_element_type=jnp.float32)

    return pl.pallas_call(
        body, out_shape=jax.ShapeDtypeStruct((_NPAD, _D), jnp.float32)
    )(x, w)


def _prep(parts, y):
    def body(p_ref, y_ref, h1_ref, n_ref):
        deg = jnp.sum(p_ref[...], axis=0)
        norm = lax.rsqrt(jnp.maximum(deg, 1.0))
        n_col = norm[:, None]
        n_ref[...] = n_col
        h1_ref[...] = y_ref[...] * n_col

    return pl.pallas_call(
        body,
        out_shape=[
            jax.ShapeDtypeStruct((_NPAD, _D), jnp.float32),
            jax.ShapeDtypeStruct((_NPAD, 1), jnp.float32),
        ],
    )(parts, y)


def _mid(zp, n_col):
    def body(z_ref, n_ref, o_ref):
        n2 = n_ref[...] * n_ref[...]
        o_ref[...] = (z_ref[0] + z_ref[1]) * n2

    return pl.pallas_call(
        body, out_shape=jax.ShapeDtypeStruct((_NPAD, _D), jnp.float32)
    )(zp, n_col)


def _final(up, n_col, b2d):
    def body(u_ref, n_ref, b_ref, o_ref):
        u = (u_ref[0] + u_ref[1]) * n_ref[...]
        o_ref[...] = u[:_N, :] + b_ref[...]

    return pl.pallas_call(
        body, out_shape=jax.ShapeDtypeStruct((_N, _D), jnp.float32)
    )(up, n_col, b2d)


def kernel(features, edge_index, W, b):
    src = edge_index[0]
    dst = edge_index[1]
    e = src.shape[0]
    fill = _N + jnp.arange(_EPAD - e, dtype=jnp.int32) % (_NPAD - _N)
    src2d = jnp.concatenate([src, fill]).reshape(_NW * _CH, 128)
    dst2d = jnp.concatenate([dst, fill]).reshape(_NW * _CH, 128)
    x_pad = jnp.pad(features, ((0, _NPAD - _N), (0, 0)))

    parts = _deg_partials(dst2d)
    y = _matmul(x_pad, W)
    h1, n_col = _prep(parts, y)
    zp = _hop(h1, src2d, dst2d)
    h2 = _mid(zp, n_col)
    up = _hop(h2, src2d, dst2d)
    return _final(up, n_col, jnp.reshape(b, (1, _D)))
